# Initial kernel scaffold; baseline (speedup 1.0000x reference)
import jax
import jax.numpy as jnp
from jax.experimental import pallas as pl


def kernel(hit_embed, hit_phi):
    raise NotImplementedError("write your pallas kernel here")


# speedup vs baseline: 3.8497x; 3.8497x over previous
import functools

import jax
import jax.numpy as jnp
import numpy as np
from jax import lax
from jax.experimental import pallas as pl
from jax.experimental.pallas import tpu as pltpu
from jax.experimental.pallas import tpu_sc as plsc

N = 100000
D = 256
L = 16
NTILES = 16
CH_S = 6400
NP = NTILES * CH_S
NV = CH_S // L
SROWS = CH_S // 128
NW = 32
CH_G = NP // NW
GCHUNK = 128
NCH = CH_G // GCHUNK

_SIGN = np.int32(-2147483648)

_mesh = plsc.VectorSubcoreMesh(core_axis_name="c", subcore_axis_name="s")


def _digits(k16, p):
    if p == 0:
        sh = k16
    else:
        sh = lax.shift_right_logical(k16, jnp.full((L,), 8 * p, jnp.int32))
    return lax.bitwise_and(sh, jnp.full((L,), 255, jnp.int32))


@functools.partial(
    pl.kernel,
    out_type=(
        jax.ShapeDtypeStruct((NP,), jnp.float32),
        jax.ShapeDtypeStruct((NP,), jnp.int32),
    ),
    mesh=_mesh,
    compiler_params=pltpu.CompilerParams(needs_layout_passes=False),
    scratch_types=[
        pltpu.VMEM((CH_S,), jnp.float32),
        pltpu.VMEM((CH_S,), jnp.int32),
        pltpu.VMEM((CH_S,), jnp.int32),
        pltpu.VMEM((SROWS, 128), jnp.int32),
        pltpu.VMEM((256,), jnp.int32),
        pltpu.VMEM((NTILES, 256), jnp.int32),
        pltpu.VMEM((256,), jnp.int32),
        pltpu.VMEM_SHARED((NP,), jnp.int32),
        pltpu.VMEM_SHARED((NP,), jnp.int32),
        pltpu.VMEM_SHARED((NP,), jnp.int32),
        pltpu.VMEM_SHARED((NP,), jnp.int32),
        pltpu.VMEM_SHARED((NTILES, 256), jnp.int32),
    ],
)
def _sort_kernel(phi_hbm, outphi_hbm, outidx_hbm, phiv, keys_v, vals_v,
                 dstidx, hist_v, histall_v, counter_v,
                 bufAk, bufAv, bufBk, bufBv, histmat):
    cid = lax.axis_index("c")
    sid = lax.axis_index("s")

    @pl.when(cid == 0)
    def _core0():
        base = sid * CH_S

        pltpu.sync_copy(phi_hbm.at[pl.ds(base, CH_S)], phiv)

        def init_body(i, carry):
            sl = pl.ds(i * L, L)
            b = plsc.bitcast(phiv[sl], jnp.int32)
            key = jnp.where(b < 0, ~b, b ^ _SIGN)
            g = base + i * L + lax.iota(jnp.int32, L)
            key = jnp.where(g < N, key, np.int32(-1))
            val = jnp.where(g < N, g, g - N)
            keys_v[sl] = key
            vals_v[sl] = val
            return carry

        lax.fori_loop(0, NV, init_body, 0)
        pltpu.sync_copy(keys_v, bufAk.at[pl.ds(base, CH_S)])
        pltpu.sync_copy(vals_v, bufAv.at[pl.ds(base, CH_S)])
        plsc.subcore_barrier()

        bufs = [(bufAk, bufAv, bufBk, bufBv), (bufBk, bufBv, bufAk, bufAv)] * 2
        for p, (srck, srcv, dstk, dstv) in enumerate(bufs):
            pltpu.sync_copy(srck.at[pl.ds(base, CH_S)], keys_v)

            def zero_body(j, carry):
                hist_v[pl.ds(j * L, L)] = jnp.zeros((L,), jnp.int32)
                return carry

            lax.fori_loop(0, 256 // L, zero_body, 0)

            def hist_body(i, carry):
                d = _digits(keys_v[pl.ds(i * L, L)], p)
                cnt, lastm = plsc.scan_count(d)
                plsc.addupdate_scatter(hist_v, [d], cnt, mask=lastm)
                return carry

            lax.fori_loop(0, NV, hist_body, 0)
            pltpu.sync_copy(hist_v, histmat.at[sid])
            plsc.subcore_barrier()
            pltpu.sync_copy(histmat, histall_v)

            def off_body(j, carry):
                sl = pl.ds(j * L, L)
                acc_tot = jnp.zeros((L,), jnp.int32)
                acc_pre = jnp.zeros((L,), jnp.int32)
                for tt in range(NTILES):
                    h = histall_v[tt, sl]
                    acc_tot = acc_tot + h
                    acc_pre = acc_pre + jnp.where(tt < sid, h, 0)
                cums = plsc.cumsum(acc_tot)
                counter_v[sl] = carry + (cums - acc_tot) + acc_pre
                return carry + jnp.sum(acc_tot)

            lax.fori_loop(0, 256 // L, off_body, np.int32(0))

            pltpu.sync_copy(srcv.at[pl.ds(base, CH_S)], vals_v)

            def rank_body(r, carry):
                for u in range(128 // L):
                    d = _digits(keys_v[pl.ds(r * 128 + u * L, L)], p)
                    cnt, lastm = plsc.scan_count(d)
                    cur = plsc.load_gather(counter_v, [d])
                    dstidx[r, pl.ds(u * L, L)] = cur + cnt - 1
                    plsc.store_scatter(counter_v, [d], cur + cnt, mask=lastm)
                return carry

            lax.fori_loop(0, SROWS, rank_body, 0)

            def scat_body(r, carry):
                idxrow = dstidx.at[r]
                pltpu.sync_copy(keys_v.at[pl.ds(r * 128, 128)], dstk.at[idxrow])
                pltpu.sync_copy(vals_v.at[pl.ds(r * 128, 128)], dstv.at[idxrow])
                return carry

            lax.fori_loop(0, SROWS, scat_body, 0)
            plsc.subcore_barrier()

        pltpu.sync_copy(bufAk.at[pl.ds(base, CH_S)], keys_v)

        def out_body(i, carry):
            sl = pl.ds(i * L, L)
            m = keys_v[sl]
            b = jnp.where(m < 0, m ^ _SIGN, ~m)
            phiv[sl] = plsc.bitcast(b, jnp.float32)
            return carry

        lax.fori_loop(0, NV, out_body, 0)
        pltpu.sync_copy(phiv, outphi_hbm.at[pl.ds(base, CH_S)])
        pltpu.sync_copy(bufAv.at[pl.ds(base, CH_S)], vals_v)
        pltpu.sync_copy(vals_v, outidx_hbm.at[pl.ds(base, CH_S)])


@functools.partial(
    pl.kernel,
    out_type=jax.ShapeDtypeStruct((NP, D), jnp.float32),
    mesh=_mesh,
    scratch_types=[
        pltpu.VMEM((CH_G,), jnp.int32),
        pltpu.VMEM((2, GCHUNK, D), jnp.float32),
        pltpu.SemaphoreType.DMA((2,)),
    ],
)
def _gather_kernel(table_hbm, idx_hbm, out_hbm, idxv, rows, sem):
    cid = lax.axis_index("c")
    sid = lax.axis_index("s")
    wid = sid * 2 + cid
    base = wid * CH_G
    pltpu.sync_copy(idx_hbm.at[pl.ds(base, CH_G)], idxv)

    pltpu.async_copy(table_hbm.at[idxv.at[pl.ds(0, GCHUNK)]], rows.at[0],
                     sem.at[0])

    def loop_body(c, carry):
        b = lax.rem(c, 2)

        @pl.when(c + 1 < NCH)
        def _():
            pltpu.async_copy(
                table_hbm.at[idxv.at[pl.ds((c + 1) * GCHUNK, GCHUNK)]],
                rows.at[1 - b], sem.at[1 - b])

        pltpu.make_async_copy(table_hbm.at[pl.ds(0, GCHUNK)], rows.at[b],
                              sem.at[b]).wait()
        pltpu.sync_copy(rows.at[b], out_hbm.at[pl.ds(base + c * GCHUNK, GCHUNK)])
        return carry

    lax.fori_loop(0, NCH, loop_body, 0)


def kernel(hit_embed, hit_phi):
    phi = hit_phi.reshape(N)
    phi_pad = jnp.pad(phi, (0, NP - N))
    phi_sorted_pad, idx_pad = _sort_kernel(phi_pad)
    table = hit_embed.reshape(N, D)
    out_pad = _gather_kernel(table, idx_pad)
    hit_embed_sorted = out_pad[:N].reshape(1, N, D)
    hit_phi_sorted = phi_sorted_pad[:N].reshape(1, N)
    return hit_embed_sorted, hit_phi_sorted


# speedup vs baseline: 5.0193x; 1.3038x over previous
import functools

import jax
import jax.numpy as jnp
import numpy as np
from jax import lax
from jax.experimental import pallas as pl
from jax.experimental.pallas import tpu as pltpu
from jax.experimental.pallas import tpu_sc as plsc

N = 100000
D = 256
L = 16
NTILES = 16
CH_S = 6400
NP = NTILES * CH_S
NV = CH_S // L
SROWS = CH_S // 128
NW = 32
CH_G = NP // NW
GCHUNK = 128
NCH = CH_G // GCHUNK

_SIGN = np.int32(-2147483648)

_mesh = plsc.VectorSubcoreMesh(core_axis_name="c", subcore_axis_name="s")


def _digits(k16, p):
    if p == 0:
        sh = k16
    else:
        sh = lax.shift_right_logical(k16, jnp.full((L,), 8 * p, jnp.int32))
    return lax.bitwise_and(sh, jnp.full((L,), 255, jnp.int32))


@functools.partial(
    pl.kernel,
    out_type=(
        jax.ShapeDtypeStruct((N,), jnp.float32),
        jax.ShapeDtypeStruct((NP,), jnp.int32),
    ),
    mesh=_mesh,
    compiler_params=pltpu.CompilerParams(needs_layout_passes=False),
    scratch_types=[
        pltpu.VMEM((CH_S,), jnp.float32),
        pltpu.VMEM((CH_S,), jnp.int32),
        pltpu.VMEM((CH_S,), jnp.int32),
        pltpu.VMEM((SROWS, 128), jnp.int32),
        pltpu.VMEM((256,), jnp.int32),
        pltpu.VMEM((NTILES, 256), jnp.int32),
        pltpu.VMEM((256,), jnp.int32),
        pltpu.VMEM_SHARED((NP,), jnp.int32),
        pltpu.VMEM_SHARED((NP,), jnp.int32),
        pltpu.VMEM_SHARED((NP,), jnp.int32),
        pltpu.VMEM_SHARED((NP,), jnp.int32),
        pltpu.VMEM_SHARED((NTILES, 256), jnp.int32),
    ],
)
def _sort_kernel(phi_hbm, outphi_hbm, outidx_hbm, phiv, keys_v, vals_v,
                 dstidx, hist_v, histall_v, counter_v,
                 bufAk, bufAv, bufBk, bufBv, histmat):
    cid = lax.axis_index("c")
    sid = lax.axis_index("s")

    @pl.when(cid == 0)
    def _core0():
        base = sid * CH_S

        pltpu.sync_copy(phi_hbm.at[pl.ds(base, CH_S)], phiv)

        def init_body(i, carry):
            sl = pl.ds(i * L, L)
            b = plsc.bitcast(phiv[sl], jnp.int32)
            key = jnp.where(b < 0, ~b, b ^ _SIGN)
            g = base + i * L + lax.iota(jnp.int32, L)
            key = jnp.where(g < N, key, np.int32(-1))
            val = jnp.where(g < N, g, g - N)
            keys_v[sl] = key
            vals_v[sl] = val
            return carry

        lax.fori_loop(0, NV, init_body, 0)
        pltpu.sync_copy(keys_v, bufAk.at[pl.ds(base, CH_S)])
        pltpu.sync_copy(vals_v, bufAv.at[pl.ds(base, CH_S)])
        plsc.subcore_barrier()

        bufs = [(bufAk, bufAv, bufBk, bufBv), (bufBk, bufBv, bufAk, bufAv)] * 2
        for p, (srck, srcv, dstk, dstv) in enumerate(bufs):
            pltpu.sync_copy(srck.at[pl.ds(base, CH_S)], keys_v)

            def zero_body(j, carry):
                hist_v[pl.ds(j * L, L)] = jnp.zeros((L,), jnp.int32)
                return carry

            lax.fori_loop(0, 256 // L, zero_body, 0)

            def hist_body(i, carry):
                d = _digits(keys_v[pl.ds(i * L, L)], p)
                cnt, lastm = plsc.scan_count(d)
                plsc.addupdate_scatter(hist_v, [d], cnt, mask=lastm)
                return carry

            lax.fori_loop(0, NV, hist_body, 0)
            pltpu.sync_copy(hist_v, histmat.at[sid])
            plsc.subcore_barrier()
            pltpu.sync_copy(histmat, histall_v)

            def off_body(j, carry):
                sl = pl.ds(j * L, L)
                acc_tot = jnp.zeros((L,), jnp.int32)
                acc_pre = jnp.zeros((L,), jnp.int32)
                for tt in range(NTILES):
                    h = histall_v[tt, sl]
                    acc_tot = acc_tot + h
                    acc_pre = acc_pre + jnp.where(tt < sid, h, 0)
                cums = plsc.cumsum(acc_tot)
                counter_v[sl] = carry + (cums - acc_tot) + acc_pre
                return carry + jnp.sum(acc_tot)

            lax.fori_loop(0, 256 // L, off_body, np.int32(0))

            pltpu.sync_copy(srcv.at[pl.ds(base, CH_S)], vals_v)

            def rank_body(r, carry):
                for u in range(128 // L):
                    d = _digits(keys_v[pl.ds(r * 128 + u * L, L)], p)
                    cnt, lastm = plsc.scan_count(d)
                    cur = plsc.load_gather(counter_v, [d])
                    dstidx[r, pl.ds(u * L, L)] = cur + cnt - 1
                    plsc.store_scatter(counter_v, [d], cur + cnt, mask=lastm)
                return carry

            lax.fori_loop(0, SROWS, rank_body, 0)

            def scat_body(r, carry):
                idxrow = dstidx.at[r]
                pltpu.sync_copy(keys_v.at[pl.ds(r * 128, 128)], dstk.at[idxrow])
                pltpu.sync_copy(vals_v.at[pl.ds(r * 128, 128)], dstv.at[idxrow])
                return carry

            lax.fori_loop(0, SROWS, scat_body, 0)
            plsc.subcore_barrier()

        pltpu.sync_copy(bufAk.at[pl.ds(base, CH_S)], keys_v)

        def out_body(i, carry):
            sl = pl.ds(i * L, L)
            m = keys_v[sl]
            b = jnp.where(m < 0, m ^ _SIGN, ~m)
            phiv[sl] = plsc.bitcast(b, jnp.float32)
            return carry

        lax.fori_loop(0, NV, out_body, 0)

        @pl.when(sid < NTILES - 1)
        def _full_phi():
            pltpu.sync_copy(phiv, outphi_hbm.at[pl.ds(base, CH_S)])

        @pl.when(sid == NTILES - 1)
        def _tail_phi():
            tail = N - (NTILES - 1) * CH_S
            pltpu.sync_copy(phiv.at[pl.ds(0, tail)],
                            outphi_hbm.at[pl.ds((NTILES - 1) * CH_S, tail)])

        pltpu.sync_copy(bufAv.at[pl.ds(base, CH_S)], vals_v)
        pltpu.sync_copy(vals_v, outidx_hbm.at[pl.ds(base, CH_S)])


@functools.partial(
    pl.kernel,
    out_type=jax.ShapeDtypeStruct((N, D), jnp.float32),
    mesh=_mesh,
    scratch_types=[
        pltpu.VMEM((CH_G,), jnp.int32),
        pltpu.VMEM((2, GCHUNK, D), jnp.float32),
        pltpu.SemaphoreType.DMA((2,)),
    ],
)
def _gather_kernel(table_hbm, idx_hbm, out_hbm, idxv, rows, sem):
    cid = lax.axis_index("c")
    sid = lax.axis_index("s")
    wid = sid * 2 + cid
    base = wid * CH_G
    pltpu.sync_copy(idx_hbm.at[pl.ds(base, CH_G)], idxv)

    pltpu.async_copy(table_hbm.at[idxv.at[pl.ds(0, GCHUNK)]], rows.at[0],
                     sem.at[0])

    def loop_body(c, carry):
        b = lax.rem(c, 2)

        @pl.when(c + 1 < NCH)
        def _():
            pltpu.async_copy(
                table_hbm.at[idxv.at[pl.ds((c + 1) * GCHUNK, GCHUNK)]],
                rows.at[1 - b], sem.at[1 - b])

        pltpu.make_async_copy(table_hbm.at[pl.ds(0, GCHUNK)], rows.at[b],
                              sem.at[b]).wait()
        start = base + c * GCHUNK

        @pl.when(start + GCHUNK <= N)
        def _full():
            pltpu.sync_copy(rows.at[b], out_hbm.at[pl.ds(start, GCHUNK)])

        for k in range(GCHUNK // 32):
            ps = start + k * 32

            @pl.when(jnp.logical_and(start + GCHUNK > N, ps + 32 <= N))
            def _piece():
                pltpu.sync_copy(rows.at[b].at[pl.ds(k * 32, 32)],
                                out_hbm.at[pl.ds(ps, 32)])
        return carry

    lax.fori_loop(0, NCH, loop_body, 0)


def kernel(hit_embed, hit_phi):
    phi = hit_phi.reshape(N)
    phi_pad = jnp.pad(phi, (0, NP - N))
    phi_sorted, idx_pad = _sort_kernel(phi_pad)
    table = hit_embed.reshape(N, D)
    out = _gather_kernel(table, idx_pad)
    return out.reshape(1, N, D), phi_sorted.reshape(1, N)


# speedup vs baseline: 5.4379x; 1.0834x over previous
import functools

import jax
import jax.numpy as jnp
import numpy as np
from jax import lax
from jax.experimental import pallas as pl
from jax.experimental.pallas import tpu as pltpu
from jax.experimental.pallas import tpu_sc as plsc

N = 100000
D = 256
L = 16
NTILES = 16
CH_S = 6400
NP = NTILES * CH_S
NV = CH_S // L
SROWS = CH_S // 128
NW = 32
CH_G = NP // NW
GCHUNK = 128
NCH = CH_G // GCHUNK

_SIGN = np.int32(-2147483648)

RBITS = 11
BINS = 1 << RBITS
NPASS = 3

_mesh = plsc.VectorSubcoreMesh(core_axis_name="c", subcore_axis_name="s")


def _digits(k16, p):
    if p == 0:
        sh = k16
    else:
        sh = lax.shift_right_logical(k16, jnp.full((L,), RBITS * p, jnp.int32))
    return lax.bitwise_and(sh, jnp.full((L,), BINS - 1, jnp.int32))


@functools.partial(
    pl.kernel,
    out_type=(
        jax.ShapeDtypeStruct((N,), jnp.float32),
        jax.ShapeDtypeStruct((NP,), jnp.int32),
    ),
    mesh=_mesh,
    compiler_params=pltpu.CompilerParams(needs_layout_passes=False),
    scratch_types=[
        pltpu.VMEM((CH_S,), jnp.float32),
        pltpu.VMEM((CH_S,), jnp.int32),
        pltpu.VMEM((CH_S,), jnp.int32),
        pltpu.VMEM((SROWS, 128), jnp.int32),
        pltpu.VMEM((BINS,), jnp.int32),
        pltpu.VMEM((NTILES, BINS), jnp.int32),
        pltpu.VMEM((BINS,), jnp.int32),
        pltpu.VMEM_SHARED((NP,), jnp.int32),
        pltpu.VMEM_SHARED((NP,), jnp.int32),
        pltpu.VMEM_SHARED((NP,), jnp.int32),
        pltpu.VMEM_SHARED((NP,), jnp.int32),
        pltpu.VMEM_SHARED((NTILES, BINS), jnp.int32),
    ],
)
def _sort_kernel(phi_hbm, outphi_hbm, outidx_hbm, phiv, keys_v, vals_v,
                 dstidx, hist_v, histall_v, counter_v,
                 bufAk, bufAv, bufBk, bufBv, histmat):
    cid = lax.axis_index("c")
    sid = lax.axis_index("s")

    @pl.when(cid == 0)
    def _core0():
        base = sid * CH_S

        @pl.when(base + CH_S <= N)
        def _ld_full():
            pltpu.sync_copy(phi_hbm.at[pl.ds(base, CH_S)], phiv)

        @pl.when(base + CH_S > N)
        def _ld_tail():
            tail = N - (NTILES - 1) * CH_S
            pltpu.sync_copy(phi_hbm.at[pl.ds((NTILES - 1) * CH_S, tail)],
                            phiv.at[pl.ds(0, tail)])

        def init_body(i, carry):
            sl = pl.ds(i * L, L)
            b = plsc.bitcast(phiv[sl], jnp.int32)
            key = jnp.where(b < 0, ~b, b ^ _SIGN)
            g = base + i * L + lax.iota(jnp.int32, L)
            key = jnp.where(g < N, key, np.int32(-1))
            val = jnp.where(g < N, g, g - N)
            keys_v[sl] = key
            vals_v[sl] = val
            return carry

        lax.fori_loop(0, NV, init_body, 0)
        pltpu.sync_copy(keys_v, bufAk.at[pl.ds(base, CH_S)])
        pltpu.sync_copy(vals_v, bufAv.at[pl.ds(base, CH_S)])
        plsc.subcore_barrier()

        bufs = ([(bufAk, bufAv, bufBk, bufBv),
                 (bufBk, bufBv, bufAk, bufAv)] * NPASS)[:NPASS]
        for p, (srck, srcv, dstk, dstv) in enumerate(bufs):
            pltpu.sync_copy(srck.at[pl.ds(base, CH_S)], keys_v)

            def zero_body(j, carry):
                hist_v[pl.ds(j * L, L)] = jnp.zeros((L,), jnp.int32)
                return carry

            lax.fori_loop(0, BINS // L, zero_body, 0)

            def hist_body(i, carry):
                d = _digits(keys_v[pl.ds(i * L, L)], p)
                cnt, lastm = plsc.scan_count(d)
                plsc.addupdate_scatter(hist_v, [d], cnt, mask=lastm)
                return carry

            lax.fori_loop(0, NV, hist_body, 0)
            pltpu.sync_copy(hist_v, histmat.at[sid])
            plsc.subcore_barrier()
            pltpu.sync_copy(histmat, histall_v)

            def off_body(j, carry):
                sl = pl.ds(j * L, L)
                acc_tot = jnp.zeros((L,), jnp.int32)
                acc_pre = jnp.zeros((L,), jnp.int32)
                for tt in range(NTILES):
                    h = histall_v[tt, sl]
                    acc_tot = acc_tot + h
                    acc_pre = acc_pre + jnp.where(tt < sid, h, 0)
                cums = plsc.cumsum(acc_tot)
                counter_v[sl] = carry + (cums - acc_tot) + acc_pre
                return carry + jnp.sum(acc_tot)

            lax.fori_loop(0, BINS // L, off_body, np.int32(0))

            pltpu.sync_copy(srcv.at[pl.ds(base, CH_S)], vals_v)

            def rank_body(r, carry):
                for u in range(128 // L):
                    d = _digits(keys_v[pl.ds(r * 128 + u * L, L)], p)
                    cnt, lastm = plsc.scan_count(d)
                    cur = plsc.load_gather(counter_v, [d])
                    dstidx[r, pl.ds(u * L, L)] = cur + cnt - 1
                    plsc.store_scatter(counter_v, [d], cur + cnt, mask=lastm)
                return carry

            lax.fori_loop(0, SROWS, rank_body, 0)

            def scat_body(r, carry):
                idxrow = dstidx.at[r]
                pltpu.sync_copy(keys_v.at[pl.ds(r * 128, 128)], dstk.at[idxrow])
                pltpu.sync_copy(vals_v.at[pl.ds(r * 128, 128)], dstv.at[idxrow])
                return carry

            lax.fori_loop(0, SROWS, scat_body, 0)
            plsc.subcore_barrier()

        fink, finv = (bufAk, bufAv) if NPASS % 2 == 0 else (bufBk, bufBv)
        pltpu.sync_copy(fink.at[pl.ds(base, CH_S)], keys_v)

        def out_body(i, carry):
            sl = pl.ds(i * L, L)
            m = keys_v[sl]
            b = jnp.where(m < 0, m ^ _SIGN, ~m)
            phiv[sl] = plsc.bitcast(b, jnp.float32)
            return carry

        lax.fori_loop(0, NV, out_body, 0)

        @pl.when(sid < NTILES - 1)
        def _full_phi():
            pltpu.sync_copy(phiv, outphi_hbm.at[pl.ds(base, CH_S)])

        @pl.when(sid == NTILES - 1)
        def _tail_phi():
            tail = N - (NTILES - 1) * CH_S
            pltpu.sync_copy(phiv.at[pl.ds(0, tail)],
                            outphi_hbm.at[pl.ds((NTILES - 1) * CH_S, tail)])

        pltpu.sync_copy(finv.at[pl.ds(base, CH_S)], vals_v)
        pltpu.sync_copy(vals_v, outidx_hbm.at[pl.ds(base, CH_S)])


@functools.partial(
    pl.kernel,
    out_type=jax.ShapeDtypeStruct((N, D), jnp.float32),
    mesh=_mesh,
    scratch_types=[
        pltpu.VMEM((CH_G,), jnp.int32),
        pltpu.VMEM((2, GCHUNK, D), jnp.float32),
        pltpu.SemaphoreType.DMA((2,)),
    ],
)
def _gather_kernel(table_hbm, idx_hbm, out_hbm, idxv, rows, sem):
    cid = lax.axis_index("c")
    sid = lax.axis_index("s")
    wid = sid * 2 + cid
    base = wid * CH_G
    pltpu.sync_copy(idx_hbm.at[pl.ds(base, CH_G)], idxv)

    pltpu.async_copy(table_hbm.at[idxv.at[pl.ds(0, GCHUNK)]], rows.at[0],
                     sem.at[0])

    def loop_body(c, carry):
        b = lax.rem(c, 2)

        @pl.when(c + 1 < NCH)
        def _():
            pltpu.async_copy(
                table_hbm.at[idxv.at[pl.ds((c + 1) * GCHUNK, GCHUNK)]],
                rows.at[1 - b], sem.at[1 - b])

        pltpu.make_async_copy(table_hbm.at[pl.ds(0, GCHUNK)], rows.at[b],
                              sem.at[b]).wait()
        start = base + c * GCHUNK

        @pl.when(start + GCHUNK <= N)
        def _full():
            pltpu.sync_copy(rows.at[b], out_hbm.at[pl.ds(start, GCHUNK)])

        for k in range(GCHUNK // 32):
            ps = start + k * 32

            @pl.when(jnp.logical_and(start + GCHUNK > N, ps + 32 <= N))
            def _piece():
                pltpu.sync_copy(rows.at[b].at[pl.ds(k * 32, 32)],
                                out_hbm.at[pl.ds(ps, 32)])
        return carry

    lax.fori_loop(0, NCH, loop_body, 0)


def kernel(hit_embed, hit_phi):
    phi = hit_phi.reshape(N)
    phi_sorted, idx_pad = _sort_kernel(phi)
    table = hit_embed.reshape(N, D)
    out = _gather_kernel(table, idx_pad)
    return out.reshape(1, N, D), phi_sorted.reshape(1, N)


# speedup vs baseline: 5.9983x; 1.1031x over previous
import functools

import jax
import jax.numpy as jnp
import numpy as np
from jax import lax
from jax.experimental import pallas as pl
from jax.experimental.pallas import tpu as pltpu
from jax.experimental.pallas import tpu_sc as plsc

N = 100000
D = 256
L = 16
NTILES = 16
CH_S = 6400
NP = NTILES * CH_S
NV = CH_S // L
SROWS = CH_S // 128
NW = 32
CH_G = NP // NW
GCHUNK = 128
NCH = CH_G // GCHUNK

_SIGN = np.int32(-2147483648)

RBITS = 11
BINS = 1 << RBITS
NPASS = 3

_mesh = plsc.VectorSubcoreMesh(core_axis_name="c", subcore_axis_name="s")


def _digits(k16, p):
    if p == 0:
        sh = k16
    else:
        sh = lax.shift_right_logical(k16, jnp.full((L,), RBITS * p, jnp.int32))
    return lax.bitwise_and(sh, jnp.full((L,), BINS - 1, jnp.int32))


@functools.partial(
    pl.kernel,
    out_type=(
        jax.ShapeDtypeStruct((N,), jnp.float32),
        jax.ShapeDtypeStruct((NP,), jnp.int32),
    ),
    mesh=_mesh,
    compiler_params=pltpu.CompilerParams(needs_layout_passes=False),
    scratch_types=[
        pltpu.VMEM((CH_S,), jnp.float32),
        pltpu.VMEM((CH_S,), jnp.int32),
        pltpu.VMEM((CH_S,), jnp.int32),
        pltpu.VMEM((SROWS, 128), jnp.int32),
        pltpu.VMEM((BINS,), jnp.int32),
        pltpu.VMEM((NTILES, BINS), jnp.int32),
        pltpu.VMEM((BINS,), jnp.int32),
        pltpu.VMEM_SHARED((NP,), jnp.int32),
        pltpu.VMEM_SHARED((NP,), jnp.int32),
        pltpu.VMEM_SHARED((NP,), jnp.int32),
        pltpu.VMEM_SHARED((NP,), jnp.int32),
        pltpu.VMEM_SHARED((NTILES, BINS), jnp.int32),
        pltpu.SemaphoreType.DMA,
    ],
)
def _sort_kernel(phi_hbm, outphi_hbm, outidx_hbm, phiv, keys_v, vals_v,
                 dstidx, hist_v, histall_v, counter_v,
                 bufAk, bufAv, bufBk, bufBv, histmat, ssem):
    cid = lax.axis_index("c")
    sid = lax.axis_index("s")

    @pl.when(cid == 0)
    def _core0():
        base = sid * CH_S

        @pl.when(base + CH_S <= N)
        def _ld_full():
            pltpu.sync_copy(phi_hbm.at[pl.ds(base, CH_S)], phiv)

        @pl.when(base + CH_S > N)
        def _ld_tail():
            tail = N - (NTILES - 1) * CH_S
            pltpu.sync_copy(phi_hbm.at[pl.ds((NTILES - 1) * CH_S, tail)],
                            phiv.at[pl.ds(0, tail)])

        def init_body(i, carry):
            sl = pl.ds(i * L, L)
            b = plsc.bitcast(phiv[sl], jnp.int32)
            key = jnp.where(b < 0, ~b, b ^ _SIGN)
            g = base + i * L + lax.iota(jnp.int32, L)
            key = jnp.where(g < N, key, np.int32(-1))
            val = jnp.where(g < N, g, g - N)
            keys_v[sl] = key
            vals_v[sl] = val
            return carry

        lax.fori_loop(0, NV, init_body, 0)

        bufs = ([(bufAk, bufAv, bufBk, bufBv),
                 (bufBk, bufBv, bufAk, bufAv)] * NPASS)[:NPASS]
        for p, (srck, srcv, dstk, dstv) in enumerate(bufs):
            if p > 0:
                pltpu.sync_copy(srck.at[pl.ds(base, CH_S)], keys_v)

            def zero_body(j, carry):
                hist_v[pl.ds(j * L, L)] = jnp.zeros((L,), jnp.int32)
                return carry

            lax.fori_loop(0, BINS // L, zero_body, 0)

            def hist_body(i, carry):
                d = _digits(keys_v[pl.ds(i * L, L)], p)
                cnt, lastm = plsc.scan_count(d)
                plsc.addupdate_scatter(hist_v, [d], cnt, mask=lastm)
                return carry

            lax.fori_loop(0, NV, hist_body, 0)
            pltpu.sync_copy(hist_v, histmat.at[sid])
            plsc.subcore_barrier()
            pltpu.sync_copy(histmat, histall_v)

            def off_body(j, carry):
                sl = pl.ds(j * L, L)
                acc_tot = jnp.zeros((L,), jnp.int32)
                acc_pre = jnp.zeros((L,), jnp.int32)
                for tt in range(NTILES):
                    h = histall_v[tt, sl]
                    acc_tot = acc_tot + h
                    acc_pre = acc_pre + jnp.where(tt < sid, h, 0)
                cums = plsc.cumsum(acc_tot)
                counter_v[sl] = carry + (cums - acc_tot) + acc_pre
                return carry + jnp.sum(acc_tot)

            lax.fori_loop(0, BINS // L, off_body, np.int32(0))

            if p > 0:
                pltpu.sync_copy(srcv.at[pl.ds(base, CH_S)], vals_v)

            def rank_body(r, carry):
                for u in range(128 // L):
                    d = _digits(keys_v[pl.ds(r * 128 + u * L, L)], p)
                    cnt, lastm = plsc.scan_count(d)
                    cur = plsc.load_gather(counter_v, [d])
                    dstidx[r, pl.ds(u * L, L)] = cur + cnt - 1
                    plsc.store_scatter(counter_v, [d], cur + cnt, mask=lastm)
                return carry

            lax.fori_loop(0, SROWS, rank_body, 0)

            GSC = 10

            def scat_group(g, carry):
                for j in range(GSC):
                    r = g * GSC + j
                    idxrow = dstidx.at[r]
                    pltpu.async_copy(keys_v.at[pl.ds(r * 128, 128)],
                                     dstk.at[idxrow], ssem)
                    pltpu.async_copy(vals_v.at[pl.ds(r * 128, 128)],
                                     dstv.at[idxrow], ssem)
                for j in range(2 * GSC):
                    pltpu.make_async_copy(keys_v.at[pl.ds(0, 128)],
                                          dstk.at[pl.ds(0, 128)], ssem).wait()
                return carry

            lax.fori_loop(0, SROWS // GSC, scat_group, 0)
            plsc.subcore_barrier()

        fink, finv = (bufAk, bufAv) if NPASS % 2 == 0 else (bufBk, bufBv)
        pltpu.sync_copy(fink.at[pl.ds(base, CH_S)], keys_v)

        def out_body(i, carry):
            sl = pl.ds(i * L, L)
            m = keys_v[sl]
            b = jnp.where(m < 0, m ^ _SIGN, ~m)
            phiv[sl] = plsc.bitcast(b, jnp.float32)
            return carry

        lax.fori_loop(0, NV, out_body, 0)

        @pl.when(sid < NTILES - 1)
        def _full_phi():
            pltpu.sync_copy(phiv, outphi_hbm.at[pl.ds(base, CH_S)])

        @pl.when(sid == NTILES - 1)
        def _tail_phi():
            tail = N - (NTILES - 1) * CH_S
            pltpu.sync_copy(phiv.at[pl.ds(0, tail)],
                            outphi_hbm.at[pl.ds((NTILES - 1) * CH_S, tail)])

        pltpu.sync_copy(finv.at[pl.ds(base, CH_S)], vals_v)
        pltpu.sync_copy(vals_v, outidx_hbm.at[pl.ds(base, CH_S)])


@functools.partial(
    pl.kernel,
    out_type=jax.ShapeDtypeStruct((N, D), jnp.float32),
    mesh=_mesh,
    scratch_types=[
        pltpu.VMEM((CH_G,), jnp.int32),
        pltpu.VMEM((2, GCHUNK, D), jnp.float32),
        pltpu.SemaphoreType.DMA((2,)),
    ],
)
def _gather_kernel(table_hbm, idx_hbm, out_hbm, idxv, rows, sem):
    cid = lax.axis_index("c")
    sid = lax.axis_index("s")
    wid = sid * 2 + cid
    base = wid * CH_G
    pltpu.sync_copy(idx_hbm.at[pl.ds(base, CH_G)], idxv)

    pltpu.async_copy(table_hbm.at[idxv.at[pl.ds(0, GCHUNK)]], rows.at[0],
                     sem.at[0])

    def loop_body(c, carry):
        b = lax.rem(c, 2)

        @pl.when(c + 1 < NCH)
        def _():
            pltpu.async_copy(
                table_hbm.at[idxv.at[pl.ds((c + 1) * GCHUNK, GCHUNK)]],
                rows.at[1 - b], sem.at[1 - b])

        pltpu.make_async_copy(table_hbm.at[pl.ds(0, GCHUNK)], rows.at[b],
                              sem.at[b]).wait()
        start = base + c * GCHUNK

        @pl.when(start + GCHUNK <= N)
        def _full():
            pltpu.sync_copy(rows.at[b], out_hbm.at[pl.ds(start, GCHUNK)])

        for k in range(GCHUNK // 32):
            ps = start + k * 32

            @pl.when(jnp.logical_and(start + GCHUNK > N, ps + 32 <= N))
            def _piece():
                pltpu.sync_copy(rows.at[b].at[pl.ds(k * 32, 32)],
                                out_hbm.at[pl.ds(ps, 32)])
        return carry

    lax.fori_loop(0, NCH, loop_body, 0)


def kernel(hit_embed, hit_phi):
    phi = hit_phi.reshape(N)
    phi_sorted, idx_pad = _sort_kernel(phi)
    table = hit_embed.reshape(N, D)
    out = _gather_kernel(table, idx_pad)
    return out.reshape(1, N, D), phi_sorted.reshape(1, N)


# speedup vs baseline: 6.0197x; 1.0036x over previous
import functools

import jax
import jax.numpy as jnp
import numpy as np
from jax import lax
from jax.experimental import pallas as pl
from jax.experimental.pallas import tpu as pltpu
from jax.experimental.pallas import tpu_sc as plsc

N = 100000
D = 256
L = 16
NTILES = 16
CH_S = 6400
NP = NTILES * CH_S
NV = CH_S // L
SROWS = CH_S // 128
NW = 32
CH_G = NP // NW
GCHUNK = 128
NCH = CH_G // GCHUNK

_SIGN = np.int32(-2147483648)

RBITS = 11
BINS = 1 << RBITS
NPASS = 3

_mesh = plsc.VectorSubcoreMesh(core_axis_name="c", subcore_axis_name="s")


def _digits(k16, p):
    if p == 0:
        sh = k16
    else:
        sh = lax.shift_right_logical(k16, jnp.full((L,), RBITS * p, jnp.int32))
    return lax.bitwise_and(sh, jnp.full((L,), BINS - 1, jnp.int32))


@functools.partial(
    pl.kernel,
    out_type=(
        jax.ShapeDtypeStruct((N,), jnp.float32),
        jax.ShapeDtypeStruct((NP,), jnp.int32),
    ),
    mesh=_mesh,
    compiler_params=pltpu.CompilerParams(needs_layout_passes=False),
    scratch_types=[
        pltpu.VMEM((CH_S,), jnp.float32),
        pltpu.VMEM((CH_S,), jnp.int32),
        pltpu.VMEM((CH_S,), jnp.int32),
        pltpu.VMEM((SROWS, 128), jnp.int32),
        pltpu.VMEM((BINS,), jnp.int32),
        pltpu.VMEM((NTILES, BINS), jnp.int32),
        pltpu.VMEM((BINS,), jnp.int32),
        pltpu.VMEM_SHARED((NP,), jnp.int32),
        pltpu.VMEM_SHARED((NP,), jnp.int32),
        pltpu.VMEM_SHARED((NP,), jnp.int32),
        pltpu.VMEM_SHARED((NP,), jnp.int32),
        pltpu.VMEM_SHARED((NTILES, BINS), jnp.int32),
        pltpu.SemaphoreType.DMA,
    ],
)
def _sort_kernel(phi_hbm, outphi_hbm, outidx_hbm, phiv, keys_v, vals_v,
                 dstidx, hist_v, histall_v, counter_v,
                 bufAk, bufAv, bufBk, bufBv, histmat, ssem):
    cid = lax.axis_index("c")
    sid = lax.axis_index("s")

    @pl.when(cid == 0)
    def _core0():
        base = sid * CH_S

        @pl.when(base + CH_S <= N)
        def _ld_full():
            pltpu.sync_copy(phi_hbm.at[pl.ds(base, CH_S)], phiv)

        @pl.when(base + CH_S > N)
        def _ld_tail():
            tail = N - (NTILES - 1) * CH_S
            pltpu.sync_copy(phi_hbm.at[pl.ds((NTILES - 1) * CH_S, tail)],
                            phiv.at[pl.ds(0, tail)])

        def init_body(i, carry):
            sl = pl.ds(i * L, L)
            b = plsc.bitcast(phiv[sl], jnp.int32)
            key = jnp.where(b < 0, ~b, b ^ _SIGN)
            g = base + i * L + lax.iota(jnp.int32, L)
            key = jnp.where(g < N, key, np.int32(-1))
            val = jnp.where(g < N, g, g - N)
            keys_v[sl] = key
            vals_v[sl] = val
            return carry

        lax.fori_loop(0, NV, init_body, 0)

        bufs = ([(bufAk, bufAv, bufBk, bufBv),
                 (bufBk, bufBv, bufAk, bufAv)] * NPASS)[:NPASS]
        for p, (srck, srcv, dstk, dstv) in enumerate(bufs):
            if p > 0:
                pltpu.sync_copy(srck.at[pl.ds(base, CH_S)], keys_v)

            def zero_body(j, carry):
                hist_v[pl.ds(j * L, L)] = jnp.zeros((L,), jnp.int32)
                return carry

            lax.fori_loop(0, BINS // L, zero_body, 0)

            def hist_body(i, carry):
                d = _digits(keys_v[pl.ds(i * L, L)], p)
                cnt, lastm = plsc.scan_count(d)
                plsc.addupdate_scatter(hist_v, [d], cnt, mask=lastm)
                return carry

            lax.fori_loop(0, NV, hist_body, 0)
            pltpu.sync_copy(hist_v, histmat.at[sid])
            plsc.subcore_barrier()
            pltpu.sync_copy(histmat, histall_v)

            def off_body(j, carry):
                sl = pl.ds(j * L, L)
                acc_tot = jnp.zeros((L,), jnp.int32)
                acc_pre = jnp.zeros((L,), jnp.int32)
                for tt in range(NTILES):
                    h = histall_v[tt, sl]
                    acc_tot = acc_tot + h
                    acc_pre = acc_pre + jnp.where(tt < sid, h, 0)
                cums = plsc.cumsum(acc_tot)
                counter_v[sl] = carry + (cums - acc_tot) + acc_pre
                return carry + jnp.sum(acc_tot)

            lax.fori_loop(0, BINS // L, off_body, np.int32(0))

            if p > 0:
                pltpu.sync_copy(srcv.at[pl.ds(base, CH_S)], vals_v)

            def rank_body(r, carry):
                for u in range(128 // L):
                    d = _digits(keys_v[pl.ds(r * 128 + u * L, L)], p)
                    cnt, lastm = plsc.scan_count(d)
                    cur = plsc.load_gather(counter_v, [d])
                    dstidx[r, pl.ds(u * L, L)] = cur + cnt - 1
                    plsc.store_scatter(counter_v, [d], cur + cnt, mask=lastm)
                return carry

            lax.fori_loop(0, SROWS, rank_body, 0)

            GSC = 10

            def scat_group(g, carry):
                for j in range(GSC):
                    r = g * GSC + j
                    idxrow = dstidx.at[r]
                    pltpu.async_copy(keys_v.at[pl.ds(r * 128, 128)],
                                     dstk.at[idxrow], ssem)
                    pltpu.async_copy(vals_v.at[pl.ds(r * 128, 128)],
                                     dstv.at[idxrow], ssem)
                for j in range(2 * GSC):
                    pltpu.make_async_copy(keys_v.at[pl.ds(0, 128)],
                                          dstk.at[pl.ds(0, 128)], ssem).wait()
                return carry

            lax.fori_loop(0, SROWS // GSC, scat_group, 0)
            plsc.subcore_barrier()

        fink, finv = (bufAk, bufAv) if NPASS % 2 == 0 else (bufBk, bufBv)
        pltpu.sync_copy(fink.at[pl.ds(base, CH_S)], keys_v)

        def out_body(i, carry):
            sl = pl.ds(i * L, L)
            m = keys_v[sl]
            b = jnp.where(m < 0, m ^ _SIGN, ~m)
            phiv[sl] = plsc.bitcast(b, jnp.float32)
            return carry

        lax.fori_loop(0, NV, out_body, 0)

        @pl.when(sid < NTILES - 1)
        def _full_phi():
            pltpu.sync_copy(phiv, outphi_hbm.at[pl.ds(base, CH_S)])

        @pl.when(sid == NTILES - 1)
        def _tail_phi():
            tail = N - (NTILES - 1) * CH_S
            pltpu.sync_copy(phiv.at[pl.ds(0, tail)],
                            outphi_hbm.at[pl.ds((NTILES - 1) * CH_S, tail)])

        pltpu.sync_copy(finv.at[pl.ds(base, CH_S)], vals_v)
        pltpu.sync_copy(vals_v, outidx_hbm.at[pl.ds(base, CH_S)])


@functools.partial(
    pl.kernel,
    out_type=jax.ShapeDtypeStruct((N, D), jnp.float32),
    mesh=_mesh,
    scratch_types=[
        pltpu.VMEM((CH_G,), jnp.int32),
        pltpu.VMEM((3, GCHUNK, D), jnp.float32),
        pltpu.SemaphoreType.DMA((3,)),
        pltpu.SemaphoreType.DMA((3,)),
    ],
)
def _gather_kernel(table_hbm, idx_hbm, out_hbm, idxv, rows, sem, wsem):
    cid = lax.axis_index("c")
    sid = lax.axis_index("s")
    wid = sid * 2 + cid
    base = wid * CH_G
    pltpu.sync_copy(idx_hbm.at[pl.ds(base, CH_G)], idxv)

    for c0 in range(2):
        pltpu.async_copy(table_hbm.at[idxv.at[pl.ds(c0 * GCHUNK, GCHUNK)]],
                         rows.at[c0], sem.at[c0])

    def loop_body(c, carry):
        b = lax.rem(c, 3)

        @pl.when(c + 2 < NCH)
        def _():
            b2 = lax.rem(c + 2, 3)

            @pl.when(jnp.logical_and(c >= 1, base + c * GCHUNK <= N))
            def _w():
                pltpu.make_async_copy(rows.at[0],
                                      out_hbm.at[pl.ds(0, GCHUNK)],
                                      wsem.at[b2]).wait()

            pltpu.async_copy(
                table_hbm.at[idxv.at[pl.ds((c + 2) * GCHUNK, GCHUNK)]],
                rows.at[b2], sem.at[b2])

        pltpu.make_async_copy(table_hbm.at[pl.ds(0, GCHUNK)], rows.at[b],
                              sem.at[b]).wait()
        start = base + c * GCHUNK

        @pl.when(start + GCHUNK <= N)
        def _full():
            pltpu.async_copy(rows.at[b], out_hbm.at[pl.ds(start, GCHUNK)],
                             wsem.at[b])

        for k in range(GCHUNK // 32):
            ps = start + k * 32

            @pl.when(jnp.logical_and(start + GCHUNK > N, ps + 32 <= N))
            def _piece():
                pltpu.sync_copy(rows.at[b].at[pl.ds(k * 32, 32)],
                                out_hbm.at[pl.ds(ps, 32)])
        return carry

    lax.fori_loop(0, NCH, loop_body, 0)

    for k in range(NCH - 3, NCH):
        @pl.when(base + k * GCHUNK + GCHUNK <= N)
        def _drain():
            pltpu.make_async_copy(rows.at[0], out_hbm.at[pl.ds(0, GCHUNK)],
                                  wsem.at[k % 3]).wait()


def kernel(hit_embed, hit_phi):
    phi = hit_phi.reshape(N)
    phi_sorted, idx_pad = _sort_kernel(phi)
    table = hit_embed.reshape(N, D)
    out = _gather_kernel(table, idx_pad)
    return out.reshape(1, N, D), phi_sorted.reshape(1, N)


# speedup vs baseline: 6.2211x; 1.0334x over previous
import functools

import jax
import jax.numpy as jnp
import numpy as np
from jax import lax
from jax.experimental import pallas as pl
from jax.experimental.pallas import tpu as pltpu
from jax.experimental.pallas import tpu_sc as plsc

N = 100000
D = 256
L = 16
NTILES = 16
CH_S = 6400
NP = NTILES * CH_S
NV = CH_S // L
SROWS = CH_S // 128
NW = 32
CH_G = NP // NW
GCHUNK = 128
NCH = CH_G // GCHUNK

_SIGN = np.int32(-2147483648)

RBITS = 11
BINS = 1 << RBITS
NPASS = 3

_mesh = plsc.VectorSubcoreMesh(core_axis_name="c", subcore_axis_name="s")


def _digits(k16, p):
    if p == 0:
        sh = k16
    else:
        sh = lax.shift_right_logical(k16, jnp.full((L,), RBITS * p, jnp.int32))
    return lax.bitwise_and(sh, jnp.full((L,), BINS - 1, jnp.int32))


@functools.partial(
    pl.kernel,
    out_type=(
        jax.ShapeDtypeStruct((N,), jnp.float32),
        jax.ShapeDtypeStruct((NP,), jnp.int32),
    ),
    mesh=_mesh,
    compiler_params=pltpu.CompilerParams(needs_layout_passes=False),
    scratch_types=[
        pltpu.VMEM((CH_S,), jnp.float32),
        pltpu.VMEM((CH_S,), jnp.int32),
        pltpu.VMEM((CH_S,), jnp.int32),
        pltpu.VMEM((SROWS, 128), jnp.int32),
        pltpu.VMEM((BINS,), jnp.int32),
        pltpu.VMEM((BINS,), jnp.int32),
        pltpu.VMEM((NTILES, BINS), jnp.int32),
        pltpu.VMEM((BINS,), jnp.int32),
        pltpu.VMEM((BINS,), jnp.int32),
        pltpu.VMEM_SHARED((NP,), jnp.int32),
        pltpu.VMEM_SHARED((NP,), jnp.int32),
        pltpu.VMEM_SHARED((NP,), jnp.int32),
        pltpu.VMEM_SHARED((NP,), jnp.int32),
        pltpu.VMEM_SHARED((NTILES, BINS), jnp.int32),
        pltpu.SemaphoreType.DMA,
    ],
)
def _sort_kernel(phi_hbm, outphi_hbm, outidx_hbm, phiv, keys_v, vals_v,
                 dstidx, hist_v, histA_v, histall_v, counter_v, counterB_v,
                 bufAk, bufAv, bufBk, bufBv, histmat, ssem):
    cid = lax.axis_index("c")
    sid = lax.axis_index("s")

    @pl.when(cid == 0)
    def _core0():
        base = sid * CH_S

        @pl.when(base + CH_S <= N)
        def _ld_full():
            pltpu.sync_copy(phi_hbm.at[pl.ds(base, CH_S)], phiv)

        @pl.when(base + CH_S > N)
        def _ld_tail():
            tail = N - (NTILES - 1) * CH_S
            pltpu.sync_copy(phi_hbm.at[pl.ds((NTILES - 1) * CH_S, tail)],
                            phiv.at[pl.ds(0, tail)])

        def init_body(i, carry):
            sl = pl.ds(i * L, L)
            b = plsc.bitcast(phiv[sl], jnp.int32)
            key = jnp.where(b < 0, ~b, b ^ _SIGN)
            g = base + i * L + lax.iota(jnp.int32, L)
            key = jnp.where(g < N, key, np.int32(-1))
            val = jnp.where(g < N, g, g - N)
            keys_v[sl] = key
            vals_v[sl] = val
            return carry

        lax.fori_loop(0, NV, init_body, 0)

        bufs = ([(bufAk, bufAv, bufBk, bufBv),
                 (bufBk, bufBv, bufAk, bufAv)] * NPASS)[:NPASS]
        for p, (srck, srcv, dstk, dstv) in enumerate(bufs):
            if p > 0:
                pltpu.sync_copy(srck.at[pl.ds(base, CH_S)], keys_v)

            def zero_body(j, carry):
                hist_v[pl.ds(j * L, L)] = jnp.zeros((L,), jnp.int32)
                return carry

            lax.fori_loop(0, BINS // L, zero_body, 0)

            def hist_body(i, carry):
                for u in range(4):
                    d = _digits(keys_v[pl.ds((4 * i + u) * L, L)], p)
                    cnt, lastm = plsc.scan_count(d)
                    plsc.addupdate_scatter(hist_v, [d], cnt, mask=lastm)
                return carry

            lax.fori_loop(0, NV // 8, hist_body, 0)

            def snap_body(j, carry):
                sl = pl.ds(j * L, L)
                histA_v[sl] = hist_v[sl]
                return carry

            lax.fori_loop(0, BINS // L, snap_body, 0)
            lax.fori_loop(NV // 8, NV // 4, hist_body, 0)
            pltpu.sync_copy(hist_v, histmat.at[sid])
            plsc.subcore_barrier()
            pltpu.sync_copy(histmat, histall_v)

            def off_body(j, carry):
                sl = pl.ds(j * L, L)
                acc_tot = jnp.zeros((L,), jnp.int32)
                acc_pre = jnp.zeros((L,), jnp.int32)
                for tt in range(NTILES):
                    h = histall_v[tt, sl]
                    acc_tot = acc_tot + h
                    acc_pre = acc_pre + jnp.where(tt < sid, h, 0)
                cums = plsc.cumsum(acc_tot)
                cbase = carry + (cums - acc_tot) + acc_pre
                counter_v[sl] = cbase
                counterB_v[sl] = cbase + histA_v[sl]
                return carry + jnp.sum(acc_tot)

            lax.fori_loop(0, BINS // L, off_body, np.int32(0))

            if p > 0:
                pltpu.sync_copy(srcv.at[pl.ds(base, CH_S)], vals_v)

            HB = CH_S // 2
            HR = SROWS // 2

            def rank_body(r, carry):
                for u in range(128 // L):
                    dA = _digits(keys_v[pl.ds(r * 128 + u * L, L)], p)
                    dB = _digits(keys_v[pl.ds(HB + r * 128 + u * L, L)], p)
                    cntA, lastA = plsc.scan_count(dA)
                    cntB, lastB = plsc.scan_count(dB)
                    curA = plsc.load_gather(counter_v, [dA])
                    curB = plsc.load_gather(counterB_v, [dB])
                    dstidx[r, pl.ds(u * L, L)] = curA + cntA - 1
                    dstidx[HR + r, pl.ds(u * L, L)] = curB + cntB - 1
                    plsc.store_scatter(counter_v, [dA], curA + cntA,
                                       mask=lastA)
                    plsc.store_scatter(counterB_v, [dB], curB + cntB,
                                       mask=lastB)
                return carry

            lax.fori_loop(0, HR, rank_body, 0)

            GSC = 10

            def scat_group(g, carry):
                for j in range(GSC):
                    r = g * GSC + j
                    idxrow = dstidx.at[r]
                    pltpu.async_copy(keys_v.at[pl.ds(r * 128, 128)],
                                     dstk.at[idxrow], ssem)
                    pltpu.async_copy(vals_v.at[pl.ds(r * 128, 128)],
                                     dstv.at[idxrow], ssem)
                for j in range(2 * GSC):
                    pltpu.make_async_copy(keys_v.at[pl.ds(0, 128)],
                                          dstk.at[pl.ds(0, 128)], ssem).wait()
                return carry

            lax.fori_loop(0, SROWS // GSC, scat_group, 0)
            plsc.subcore_barrier()

        fink, finv = (bufAk, bufAv) if NPASS % 2 == 0 else (bufBk, bufBv)
        pltpu.sync_copy(fink.at[pl.ds(base, CH_S)], keys_v)

        def out_body(i, carry):
            sl = pl.ds(i * L, L)
            m = keys_v[sl]
            b = jnp.where(m < 0, m ^ _SIGN, ~m)
            phiv[sl] = plsc.bitcast(b, jnp.float32)
            return carry

        lax.fori_loop(0, NV, out_body, 0)

        @pl.when(sid < NTILES - 1)
        def _full_phi():
            pltpu.sync_copy(phiv, outphi_hbm.at[pl.ds(base, CH_S)])

        @pl.when(sid == NTILES - 1)
        def _tail_phi():
            tail = N - (NTILES - 1) * CH_S
            pltpu.sync_copy(phiv.at[pl.ds(0, tail)],
                            outphi_hbm.at[pl.ds((NTILES - 1) * CH_S, tail)])

        pltpu.sync_copy(finv.at[pl.ds(base, CH_S)], vals_v)
        pltpu.sync_copy(vals_v, outidx_hbm.at[pl.ds(base, CH_S)])


@functools.partial(
    pl.kernel,
    out_type=jax.ShapeDtypeStruct((N, D), jnp.float32),
    mesh=_mesh,
    scratch_types=[
        pltpu.VMEM((CH_G,), jnp.int32),
        pltpu.VMEM((3, GCHUNK, D), jnp.float32),
        pltpu.SemaphoreType.DMA((3,)),
        pltpu.SemaphoreType.DMA((3,)),
    ],
)
def _gather_kernel(table_hbm, idx_hbm, out_hbm, idxv, rows, sem, wsem):
    cid = lax.axis_index("c")
    sid = lax.axis_index("s")
    wid = sid * 2 + cid
    base = wid * CH_G
    pltpu.sync_copy(idx_hbm.at[pl.ds(base, CH_G)], idxv)

    for c0 in range(2):
        pltpu.async_copy(table_hbm.at[idxv.at[pl.ds(c0 * GCHUNK, GCHUNK)]],
                         rows.at[c0], sem.at[c0])

    def loop_body(c, carry):
        b = lax.rem(c, 3)

        @pl.when(c + 2 < NCH)
        def _():
            b2 = lax.rem(c + 2, 3)

            @pl.when(jnp.logical_and(c >= 1, base + c * GCHUNK <= N))
            def _w():
                pltpu.make_async_copy(rows.at[0],
                                      out_hbm.at[pl.ds(0, GCHUNK)],
                                      wsem.at[b2]).wait()

            pltpu.async_copy(
                table_hbm.at[idxv.at[pl.ds((c + 2) * GCHUNK, GCHUNK)]],
                rows.at[b2], sem.at[b2])

        pltpu.make_async_copy(table_hbm.at[pl.ds(0, GCHUNK)], rows.at[b],
                              sem.at[b]).wait()
        start = base + c * GCHUNK

        @pl.when(start + GCHUNK <= N)
        def _full():
            pltpu.async_copy(rows.at[b], out_hbm.at[pl.ds(start, GCHUNK)],
                             wsem.at[b])

        for k in range(GCHUNK // 32):
            ps = start + k * 32

            @pl.when(jnp.logical_and(start + GCHUNK > N, ps + 32 <= N))
            def _piece():
                pltpu.sync_copy(rows.at[b].at[pl.ds(k * 32, 32)],
                                out_hbm.at[pl.ds(ps, 32)])
        return carry

    lax.fori_loop(0, NCH, loop_body, 0)

    for k in range(NCH - 3, NCH):
        @pl.when(base + k * GCHUNK + GCHUNK <= N)
        def _drain():
            pltpu.make_async_copy(rows.at[0], out_hbm.at[pl.ds(0, GCHUNK)],
                                  wsem.at[k % 3]).wait()


def kernel(hit_embed, hit_phi):
    phi = hit_phi.reshape(N)
    phi_sorted, idx_pad = _sort_kernel(phi)
    table = hit_embed.reshape(N, D)
    out = _gather_kernel(table, idx_pad)
    return out.reshape(1, N, D), phi_sorted.reshape(1, N)
